# baseline (speedup 1.0000x reference)
import jax
import jax.numpy as jnp
from jax import lax
from jax.experimental import pallas as pl
from jax.experimental.pallas import tpu as pltpu
from jax.experimental.pallas import tpu_sc as plsc

VOCAB = 100000
DIM = 64
BATCH = 1024

_SC_INFO = plsc.get_sparse_core_info()
_NC = _SC_INFO.num_cores
_NS = _SC_INFO.num_subcores
_NW = _NC * _NS
_BPW = BATCH // _NW

_VT = 2048


def _sc_gather_body(idx_hbm, table_hbm, x_hbm, idx_v, rows_v, sem):
    wid = lax.axis_index("s") * _NC + lax.axis_index("c")
    base = wid * _BPW
    pltpu.sync_copy(idx_hbm.at[pl.ds(base, _BPW)], idx_v)
    pltpu.async_copy(table_hbm.at[idx_v], rows_v, sem).wait()
    pltpu.sync_copy(rows_v, x_hbm.at[pl.ds(base, _BPW)])


def _sc_gather(input_ids, embed_table):
    mesh = plsc.VectorSubcoreMesh(core_axis_name="c", subcore_axis_name="s")
    fn = pl.kernel(
        _sc_gather_body,
        mesh=mesh,
        compiler_params=pltpu.CompilerParams(use_tc_tiling_on_sc=False),
        out_type=jax.ShapeDtypeStruct((BATCH, DIM), jnp.float32),
        scratch_types=[
            pltpu.VMEM((_BPW,), jnp.int32),
            pltpu.VMEM((_BPW, DIM), jnp.float32),
            pltpu.SemaphoreType.DMA,
        ],
    )
    return fn(input_ids, embed_table)


_VTM = 1024
_NFULL = VOCAB // _VTM
_TAIL = VOCAB - _NFULL * _VTM
_GRID = _NFULL + 1
_NBUF = 8


def _tc_matmul_body(x_ref, w_ref, out_hbm, state_ref, bufs, tail_buf,
                    sems, tail_sem):
    i = pl.program_id(0)
    slot = lax.rem(i, _NBUF)

    @pl.when(i == 0)
    def _():
        state_ref[...] = jnp.sum(
            x_ref[...], axis=1, keepdims=True) * jnp.float32(1.0 / DIM)

    @pl.when(i >= _NBUF)
    def _():
        j = i - _NBUF
        pltpu.make_async_copy(
            bufs.at[slot],
            out_hbm.at[:, pl.ds(j * _VTM, _VTM)],
            sems.at[slot],
        ).wait()

    @pl.when(i < _NFULL)
    def _():
        bufs[slot] = lax.dot_general(
            x_ref[...], w_ref[...],
            (((1,), (1,)), ((), ())),
            preferred_element_type=jnp.float32,
        )
        pltpu.make_async_copy(
            bufs.at[slot],
            out_hbm.at[:, pl.ds(i * _VTM, _VTM)],
            sems.at[slot],
        ).start()

    @pl.when(i == _NFULL)
    def _():
        tail_buf[...] = lax.dot_general(
            x_ref[...], w_ref[pl.ds(0, _TAIL), :],
            (((1,), (1,)), ((), ())),
            preferred_element_type=jnp.float32,
        )
        pltpu.make_async_copy(
            tail_buf,
            out_hbm.at[:, pl.ds(_NFULL * _VTM, _TAIL)],
            tail_sem,
        ).start()
        for j in range(_NFULL - _NBUF + 1, _NFULL):
            s = j % _NBUF
            pltpu.make_async_copy(
                bufs.at[s],
                out_hbm.at[:, pl.ds(j * _VTM, _VTM)],
                sems.at[s],
            ).wait()
        pltpu.make_async_copy(
            tail_buf,
            out_hbm.at[:, pl.ds(_NFULL * _VTM, _TAIL)],
            tail_sem,
        ).wait()


def _tc_matmul(x, lm_head_w):
    return pl.pallas_call(
        _tc_matmul_body,
        grid=(_GRID,),
        in_specs=[
            pl.BlockSpec((BATCH, DIM), lambda i: (0, 0)),
            pl.BlockSpec((_VTM, DIM), lambda i: (i, 0)),
        ],
        out_specs=[
            pl.BlockSpec(memory_space=pl.ANY),
            pl.BlockSpec((BATCH, 1), lambda i: (0, 0)),
        ],
        out_shape=[
            jax.ShapeDtypeStruct((BATCH, VOCAB), jnp.float32),
            jax.ShapeDtypeStruct((BATCH, 1), jnp.float32),
        ],
        scratch_shapes=[
            pltpu.VMEM((_NBUF, BATCH, _VTM), jnp.float32),
            pltpu.VMEM((BATCH, _TAIL), jnp.float32),
            pltpu.SemaphoreType.DMA((_NBUF,)),
            pltpu.SemaphoreType.DMA,
        ],
    )(x, lm_head_w)


def kernel(input_ids, embed_table, lm_head_w):
    ids = input_ids.astype(jnp.int32)
    x = _sc_gather(ids, embed_table)
    logits, state = _tc_matmul(x, lm_head_w)
    return (logits, state.reshape(BATCH))


# speedup vs baseline: 2.7753x; 2.7753x over previous
import jax
import jax.numpy as jnp
from jax import lax
from jax.experimental import pallas as pl
from jax.experimental.pallas import tpu as pltpu
from jax.experimental.pallas import tpu_sc as plsc

VOCAB = 100000
DIM = 64
BATCH = 1024

_SC_INFO = plsc.get_sparse_core_info()
_NC = _SC_INFO.num_cores
_NS = _SC_INFO.num_subcores
_NW = _NC * _NS
_BPW = BATCH // _NW


def _sc_gather_body(idx_hbm, table_hbm, x_hbm, idx_v, rows_v, sem):
    wid = lax.axis_index("s") * _NC + lax.axis_index("c")
    base = wid * _BPW
    pltpu.sync_copy(idx_hbm.at[pl.ds(base, _BPW)], idx_v)
    pltpu.async_copy(table_hbm.at[idx_v], rows_v, sem).wait()
    pltpu.sync_copy(rows_v, x_hbm.at[pl.ds(base, _BPW)])


def _sc_gather(input_ids, embed_table):
    mesh = plsc.VectorSubcoreMesh(core_axis_name="c", subcore_axis_name="s")
    fn = pl.kernel(
        _sc_gather_body,
        mesh=mesh,
        compiler_params=pltpu.CompilerParams(use_tc_tiling_on_sc=False),
        out_type=jax.ShapeDtypeStruct((BATCH, DIM), jnp.float32),
        scratch_types=[
            pltpu.VMEM((_BPW,), jnp.int32),
            pltpu.VMEM((_BPW, DIM), jnp.float32),
            pltpu.SemaphoreType.DMA,
        ],
    )
    return fn(input_ids, embed_table)


_VTM = 1024
_NFULL = VOCAB // _VTM
_TAIL = VOCAB - _NFULL * _VTM
_GRID = _NFULL + 1
_NBUF = 6


def _tc_matmul_body(wt_ref, xt_ref, out_hbm, state_ref, bufs, sems):
    i = pl.program_id(0)
    slot = lax.rem(i, _NBUF)

    @pl.when(i == 0)
    def _():
        state_ref[...] = jnp.sum(
            xt_ref[...], axis=0, keepdims=True) * jnp.float32(1.0 / DIM)

    @pl.when(i >= _NBUF)
    def _():
        j = i - _NBUF
        pltpu.make_async_copy(
            bufs.at[slot],
            out_hbm.at[pl.ds(j * _VTM, _VTM), :],
            sems.at[slot],
        ).wait()

    bufs[slot] = lax.dot_general(
        wt_ref[...], xt_ref[...],
        (((0,), (0,)), ((), ())),
        preferred_element_type=jnp.float32,
    )

    @pl.when(i < _NFULL)
    def _():
        pltpu.make_async_copy(
            bufs.at[slot],
            out_hbm.at[pl.ds(i * _VTM, _VTM), :],
            sems.at[slot],
        ).start()

    @pl.when(i == _NFULL)
    def _():
        pltpu.make_async_copy(
            bufs.at[slot, pl.ds(0, _TAIL), :],
            out_hbm.at[pl.ds(_NFULL * _VTM, _TAIL), :],
            sems.at[slot],
        ).start()
        for j in range(_NFULL - _NBUF + 1, _NFULL):
            s = j % _NBUF
            pltpu.make_async_copy(
                bufs.at[s],
                out_hbm.at[pl.ds(j * _VTM, _VTM), :],
                sems.at[s],
            ).wait()
        s = _NFULL % _NBUF
        pltpu.make_async_copy(
            bufs.at[s, pl.ds(0, _TAIL), :],
            out_hbm.at[pl.ds(_NFULL * _VTM, _TAIL), :],
            sems.at[s],
        ).wait()


def _tc_matmul(xt, wt):
    return pl.pallas_call(
        _tc_matmul_body,
        grid=(_GRID,),
        in_specs=[
            pl.BlockSpec((DIM, _VTM), lambda i: (0, i)),
            pl.BlockSpec((DIM, BATCH), lambda i: (0, 0)),
        ],
        out_specs=[
            pl.BlockSpec(memory_space=pl.ANY),
            pl.BlockSpec((1, BATCH), lambda i: (0, 0)),
        ],
        out_shape=[
            jax.ShapeDtypeStruct((VOCAB, BATCH), jnp.float32),
            jax.ShapeDtypeStruct((1, BATCH), jnp.float32),
        ],
        scratch_shapes=[
            pltpu.VMEM((_NBUF, _VTM, BATCH), jnp.float32),
            pltpu.SemaphoreType.DMA((_NBUF,)),
        ],
    )(wt, xt)


def kernel(input_ids, embed_table, lm_head_w):
    ids = input_ids.astype(jnp.int32)
    x = _sc_gather(ids, embed_table)
    logits_t, state = _tc_matmul(x.T, lm_head_w.T)
    return (logits_t.T, state.reshape(BATCH))


# speedup vs baseline: 3.3568x; 1.2095x over previous
import jax
import jax.numpy as jnp
from jax import lax
from jax.experimental import pallas as pl
from jax.experimental.pallas import tpu as pltpu
from jax.experimental.pallas import tpu_sc as plsc

VOCAB = 100000
DIM = 64
BATCH = 1024

_SC_INFO = plsc.get_sparse_core_info()
_NC = _SC_INFO.num_cores
_NS = _SC_INFO.num_subcores
_NW = _NC * _NS
_BPW = BATCH // _NW


_KPAD = 128
_PT = 8192


def _pretranspose_body(wt_ref, out_ref):
    r = lax.broadcasted_iota(jnp.int32, (DIM, _KPAD), 0)
    c = lax.broadcasted_iota(jnp.int32, (DIM, _KPAD), 1)
    eye_pad = jnp.where(r == c, jnp.float32(1.0), jnp.float32(0.0))
    out_ref[...] = lax.dot_general(
        wt_ref[...], eye_pad,
        (((0,), (0,)), ((), ())),
        preferred_element_type=jnp.float32,
    )


def _pretranspose(wt):
    return pl.pallas_call(
        _pretranspose_body,
        grid=(pl.cdiv(VOCAB, _PT),),
        in_specs=[pl.BlockSpec((DIM, _PT), lambda i: (0, i))],
        out_specs=pl.BlockSpec((_PT, _KPAD), lambda i: (i, 0)),
        out_shape=jax.ShapeDtypeStruct((VOCAB, _KPAD), jnp.float32),
    )(wt)


def _sc_gather_body(idx_hbm, table_hbm, x_hbm, idx_v, rows_v, sem):
    wid = lax.axis_index("s") * _NC + lax.axis_index("c")
    base = wid * _BPW
    pltpu.sync_copy(idx_hbm.at[pl.ds(base, _BPW)], idx_v)
    pltpu.async_copy(table_hbm.at[idx_v], rows_v, sem).wait()
    pltpu.sync_copy(rows_v, x_hbm.at[pl.ds(base, _BPW)])


def _sc_gather(input_ids, table_pad):
    mesh = plsc.VectorSubcoreMesh(core_axis_name="c", subcore_axis_name="s")
    fn = pl.kernel(
        _sc_gather_body,
        mesh=mesh,
        out_type=jax.ShapeDtypeStruct((BATCH, _KPAD), jnp.float32),
        scratch_types=[
            pltpu.VMEM((_BPW,), jnp.int32),
            pltpu.VMEM((_BPW, _KPAD), jnp.float32),
            pltpu.SemaphoreType.DMA,
        ],
    )
    return fn(input_ids, table_pad)


_VTM = 1024
_NFULL = VOCAB // _VTM
_TAIL = VOCAB - _NFULL * _VTM
_GRID = _NFULL + 1
_NBUF = 6


def _tc_matmul_body(wt_ref, xt_ref, out_hbm, state_ref, bufs, sems):
    i = pl.program_id(0)
    slot = lax.rem(i, _NBUF)

    @pl.when(i == 0)
    def _():
        state_ref[...] = jnp.sum(
            xt_ref[pl.ds(0, DIM), :], axis=0,
            keepdims=True) * jnp.float32(1.0 / DIM)

    @pl.when(i >= _NBUF)
    def _():
        j = i - _NBUF
        pltpu.make_async_copy(
            bufs.at[slot],
            out_hbm.at[pl.ds(j * _VTM, _VTM), :],
            sems.at[slot],
        ).wait()

    bufs[slot] = lax.dot_general(
        wt_ref[...], xt_ref[pl.ds(0, DIM), :],
        (((0,), (0,)), ((), ())),
        preferred_element_type=jnp.float32,
    )

    @pl.when(i < _NFULL)
    def _():
        pltpu.make_async_copy(
            bufs.at[slot],
            out_hbm.at[pl.ds(i * _VTM, _VTM), :],
            sems.at[slot],
        ).start()

    @pl.when(i == _NFULL)
    def _():
        pltpu.make_async_copy(
            bufs.at[slot, pl.ds(0, _TAIL), :],
            out_hbm.at[pl.ds(_NFULL * _VTM, _TAIL), :],
            sems.at[slot],
        ).start()
        for j in range(_NFULL - _NBUF + 1, _NFULL):
            s = j % _NBUF
            pltpu.make_async_copy(
                bufs.at[s],
                out_hbm.at[pl.ds(j * _VTM, _VTM), :],
                sems.at[s],
            ).wait()
        s = _NFULL % _NBUF
        pltpu.make_async_copy(
            bufs.at[s, pl.ds(0, _TAIL), :],
            out_hbm.at[pl.ds(_NFULL * _VTM, _TAIL), :],
            sems.at[s],
        ).wait()


def _tc_matmul(xt, wt):
    return pl.pallas_call(
        _tc_matmul_body,
        grid=(_GRID,),
        in_specs=[
            pl.BlockSpec((DIM, _VTM), lambda i: (0, i)),
            pl.BlockSpec((_KPAD, BATCH), lambda i: (0, 0)),
        ],
        out_specs=[
            pl.BlockSpec(memory_space=pl.ANY),
            pl.BlockSpec((1, BATCH), lambda i: (0, 0)),
        ],
        out_shape=[
            jax.ShapeDtypeStruct((VOCAB, BATCH), jnp.float32),
            jax.ShapeDtypeStruct((1, BATCH), jnp.float32),
        ],
        scratch_shapes=[
            pltpu.VMEM((_NBUF, _VTM, BATCH), jnp.float32),
            pltpu.SemaphoreType.DMA((_NBUF,)),
        ],
    )(wt, xt)


def kernel(input_ids, embed_table, lm_head_w):
    ids = input_ids.astype(jnp.int32)
    table_pad = _pretranspose(embed_table.T)
    xpad = _sc_gather(ids, table_pad)
    logits_t, state = _tc_matmul(xpad.T, lm_head_w.T)
    return (logits_t.T, state.reshape(BATCH))
